# Initial kernel scaffold; baseline (speedup 1.0000x reference)
import jax
import jax.numpy as jnp
from jax.experimental import pallas as pl


def kernel(u2i, i2u, x_user, x_item, user_w_q, user_w_k, user_w_v, item_w_q, item_w_k, item_w_v):
    raise NotImplementedError("write your pallas kernel here")


# speedup vs baseline: 3.2024x; 3.2024x over previous
import functools

import jax
import jax.numpy as jnp
from jax import lax
from jax.experimental import pallas as pl
from jax.experimental.pallas import tpu as pltpu
from jax.experimental.pallas import tpu_sc as plsc

N = 10000
E = 320000
D = 128
DH = 32
H = 4
NPAD = 10240
NW = 32
NT = 16
EW = E // NW
ES = E // NT
C = 128
UROWS = NPAD // NT
MFLAT = NPAD * H
MSL = MFLAT // NW
NEG = -1e30
A_TAIL = EW - (EW // C) * C
B_TAIL = ES - (ES // C) * C

_mesh = plsc.VectorSubcoreMesh(core_axis_name="c", subcore_axis_name="s")
_iota16 = lambda: lax.iota(jnp.int32, 16)
_sc_params = pltpu.CompilerParams(needs_layout_passes=False)


def _zero_rows(ref, nrows, ncols):
    z = jnp.zeros((16,), jnp.float32)

    def body(r, carry):
        for j in range(ncols // 16):
            ref[r, pl.ds(j * 16, 16)] = z
        return carry

    lax.fori_loop(0, nrows, body, None)


def _fill_flat(ref, n, value):
    v = jnp.full((16,), value, jnp.float32)

    def body(i, carry):
        ref[pl.ds(i * 16, 16)] = v
        return carry

    lax.fori_loop(0, n // 16, body, None)


@functools.partial(
    pl.kernel,
    out_type=(
        jax.ShapeDtypeStruct((H * E,), jnp.float32),
        jax.ShapeDtypeStruct((NW * MFLAT,), jnp.float32),
    ),
    mesh=_mesh,
    compiler_params=_sc_params,
    scratch_types=[
        pltpu.VMEM((C,), jnp.int32),
        pltpu.VMEM((C,), jnp.int32),
        pltpu.VMEM((C, D), jnp.float32),
        pltpu.VMEM((C, D), jnp.float32),
        pltpu.VMEM((H, C), jnp.float32),
        pltpu.VMEM((MFLAT,), jnp.float32),
        pltpu.SemaphoreType.DMA,
        pltpu.SemaphoreType.DMA,
    ],
)
def _pass_a1(q_hbm, k_hbm, src_hbm, dst_hbm, l_hbm, mp_hbm,
             idx_d, idx_s, qbuf, kbuf, lbuf, m_tile, sem0, sem1):
    cid = lax.axis_index("c")
    sid = lax.axis_index("s")
    wid = cid * NT + sid

    _fill_flat(m_tile, MFLAT, NEG)
    iota = _iota16()

    def do_chunk(base, csz):
        ng = csz // 16
        pltpu.sync_copy(dst_hbm.at[pl.ds(base, csz)], idx_d.at[pl.ds(0, csz)])
        pltpu.sync_copy(src_hbm.at[pl.ds(base, csz)], idx_s.at[pl.ds(0, csz)])
        cp0 = pltpu.async_copy(q_hbm.at[idx_d], qbuf, sem0)
        cp1 = pltpu.async_copy(k_hbm.at[idx_s], kbuf, sem1)
        cp0.wait()
        cp1.wait()

        def group(g, carry):
            rows = g * 16 + iota
            dstv = idx_d[pl.ds(g * 16, 16)]
            accs = [jnp.zeros((16,), jnp.float32) for _ in range(H)]
            for d in range(D):
                cols = jnp.full((16,), d, jnp.int32)
                qv = plsc.load_gather(qbuf, [rows, cols])
                kv = plsc.load_gather(kbuf, [rows, cols])
                accs[d // DH] = accs[d // DH] + qv * kv
            for h in range(H):
                plsc.store_scatter(lbuf, [jnp.full((16,), h, jnp.int32), rows],
                                   accs[h])
                flat = dstv * H + h
                old = plsc.load_gather(m_tile, [flat])
                v = jnp.maximum(old, accs[h])
                sv, sf = plsc.sort_key_val(v, flat, descending=False)
                unused_cnt, last = plsc.scan_count(sf)
                plsc.store_scatter(m_tile, [sf], sv, mask=last)
            return carry

        lax.fori_loop(0, ng, group, None)
        for h in range(H):
            pltpu.sync_copy(lbuf.at[h, pl.ds(0, csz)],
                            l_hbm.at[pl.ds(h * E + base, csz)])

    def chunk_body(i, carry):
        do_chunk(wid * EW + i * C, C)
        return carry

    lax.fori_loop(0, EW // C, chunk_body, None)
    do_chunk(wid * EW + (EW // C) * C, A_TAIL)

    pltpu.sync_copy(m_tile, mp_hbm.at[pl.ds(wid * MFLAT, MFLAT)])


def _make_combine(op):
    @functools.partial(
        pl.kernel,
        out_type=jax.ShapeDtypeStruct((MFLAT,), jnp.float32),
        mesh=_mesh,
        compiler_params=_sc_params,
        scratch_types=[
            pltpu.VMEM((MSL,), jnp.float32),
            pltpu.VMEM((MSL,), jnp.float32),
        ],
    )
    def combine(mp_hbm, out_hbm, acc, tmp):
        cid = lax.axis_index("c")
        sid = lax.axis_index("s")
        wid = cid * NT + sid
        off = wid * MSL

        pltpu.sync_copy(mp_hbm.at[pl.ds(off, MSL)], acc)
        for t in range(1, NW):
            pltpu.sync_copy(mp_hbm.at[pl.ds(t * MFLAT + off, MSL)], tmp)

            def body(i, carry):
                sl = pl.ds(i * 16, 16)
                acc[sl] = op(acc[sl], tmp[sl])
                return carry

            lax.fori_loop(0, MSL // 16, body, None)
        pltpu.sync_copy(acc, out_hbm.at[pl.ds(off, MSL)])

    return combine


_combine_max = _make_combine(jnp.maximum)
_combine_sum = _make_combine(lambda a, b: a + b)


@functools.partial(
    pl.kernel,
    out_type=(
        jax.ShapeDtypeStruct((H * E,), jnp.float32),
        jax.ShapeDtypeStruct((NW * MFLAT,), jnp.float32),
    ),
    mesh=_mesh,
    compiler_params=_sc_params,
    scratch_types=[
        pltpu.VMEM((C,), jnp.int32),
        pltpu.VMEM((H, C), jnp.float32),
        pltpu.VMEM((H, C), jnp.float32),
        pltpu.VMEM((MFLAT,), jnp.float32),
        pltpu.VMEM((MFLAT,), jnp.float32),
    ],
)
def _pass_a2(l_hbm, m_hbm, dst_hbm, e_hbm, sp_hbm,
             idx_d, lbuf, ebuf, m0, s_tile):
    cid = lax.axis_index("c")
    sid = lax.axis_index("s")
    wid = cid * NT + sid

    pltpu.sync_copy(m_hbm, m0)
    _fill_flat(s_tile, MFLAT, 0.0)

    iota = _iota16()

    def accumulate(flat, ev):
        def cond(mask):
            return jnp.any(mask)

        def body(mask):
            unused_cnt, last = plsc.scan_count(flat, mask)
            commit = jnp.logical_and(mask, last)
            old = plsc.load_gather(s_tile, [flat])
            plsc.store_scatter(s_tile, [flat], old + ev, mask=commit)
            return jnp.logical_and(mask, jnp.logical_not(commit))

        lax.while_loop(cond, body, jnp.full((16,), True, jnp.bool_))

    def do_chunk(base, csz):
        ng = csz // 16
        pltpu.sync_copy(dst_hbm.at[pl.ds(base, csz)], idx_d.at[pl.ds(0, csz)])
        for h in range(H):
            pltpu.sync_copy(l_hbm.at[pl.ds(h * E + base, csz)],
                            lbuf.at[h, pl.ds(0, csz)])

        def group(g, carry):
            rows = g * 16 + iota
            dstv = idx_d[pl.ds(g * 16, 16)]
            for h in range(H):
                hv = jnp.full((16,), h, jnp.int32)
                lv = plsc.load_gather(lbuf, [hv, rows])
                flat = dstv * H + h
                mv = plsc.load_gather(m0, [flat])
                ev = jnp.exp(lv - mv)
                plsc.store_scatter(ebuf, [hv, rows], ev)
                accumulate(flat, ev)
            return carry

        lax.fori_loop(0, ng, group, None)
        for h in range(H):
            pltpu.sync_copy(ebuf.at[h, pl.ds(0, csz)],
                            e_hbm.at[pl.ds(h * E + base, csz)])

    def chunk_body(i, carry):
        do_chunk(wid * EW + i * C, C)
        return carry

    lax.fori_loop(0, EW // C, chunk_body, None)
    do_chunk(wid * EW + (EW // C) * C, A_TAIL)

    pltpu.sync_copy(s_tile, sp_hbm.at[pl.ds(wid * MFLAT, MFLAT)])


@functools.partial(
    pl.kernel,
    out_type=(
        jax.ShapeDtypeStruct((H, NPAD, D), jnp.float32),
        jax.ShapeDtypeStruct((H, NPAD, D), jnp.float32),
    ),
    mesh=_mesh,
    compiler_params=_sc_params,
    scratch_types=[
        pltpu.VMEM((C,), jnp.int32),
        pltpu.VMEM((C,), jnp.int32),
        pltpu.VMEM((B_TAIL,), jnp.int32),
        pltpu.VMEM((C,), jnp.float32),
        pltpu.VMEM((C, D), jnp.float32),
        pltpu.VMEM((C, D), jnp.float32),
        pltpu.VMEM_SHARED((NPAD, D), jnp.float32),
        pltpu.SemaphoreType.DMA,
    ],
)
def _pass_b(x0_hbm, src0_hbm, dst0_hbm, e0_hbm,
            x1_hbm, src1_hbm, dst1_hbm, e1_hbm, u0_hbm, u1_hbm,
            idx_s, idx_d, idx_dt, ebuf, xbuf, obuf, u_shared, sem0):
    cid = lax.axis_index("c")
    sid = lax.axis_index("s")
    iota = _iota16()

    for x_hbm, src_hbm, dst_hbm, e_hbm, u_hbm in (
            (x0_hbm, src0_hbm, dst0_hbm, e0_hbm, u0_hbm),
            (x1_hbm, src1_hbm, dst1_hbm, e1_hbm, u1_hbm)):
        for ph in range(2):
            h = 2 * ph + cid
            _zero_rows(obuf, C, D)
            for z in range(UROWS // C):
                pltpu.sync_copy(obuf,
                                u_shared.at[pl.ds(sid * UROWS + z * C, C)])
            plsc.subcore_barrier()

            def do_chunk(base, csz, dref, x_hbm=x_hbm, src_hbm=src_hbm,
                         dst_hbm=dst_hbm, e_hbm=e_hbm, h=h):
                ng = csz // 16
                pltpu.sync_copy(src_hbm.at[pl.ds(base, csz)],
                                idx_s.at[pl.ds(0, csz)])
                pltpu.sync_copy(dst_hbm.at[pl.ds(base, csz)], dref)
                pltpu.sync_copy(e_hbm.at[pl.ds(h * E + base, csz)],
                                ebuf.at[pl.ds(0, csz)])
                pltpu.async_copy(x_hbm.at[idx_s], xbuf, sem0).wait()

                def group(g, carry):
                    rows = g * 16 + iota
                    wv = ebuf[pl.ds(g * 16, 16)]
                    for d in range(D):
                        cols = jnp.full((16,), d, jnp.int32)
                        xv = plsc.load_gather(xbuf, [rows, cols])
                        plsc.store_scatter(obuf, [rows, cols], xv * wv)
                    return carry

                lax.fori_loop(0, ng, group, None)

            def chunk_body(i, carry):
                do_chunk(sid * ES + i * C, C, idx_d)
                pltpu.sync_copy(obuf, u_shared.at[idx_d], add=True)
                return carry

            lax.fori_loop(0, ES // C, chunk_body, None)
            do_chunk(sid * ES + (ES // C) * C, B_TAIL, idx_dt)
            pltpu.sync_copy(obuf.at[pl.ds(0, B_TAIL)], u_shared.at[idx_dt],
                            add=True)

            plsc.subcore_barrier()
            for z in range(UROWS // C):
                r = sid * UROWS + z * C
                pltpu.sync_copy(u_shared.at[pl.ds(r, C)],
                                u_hbm.at[h, pl.ds(r, C)])
            plsc.subcore_barrier()


def _qk_body(xu_ref, xi_ref, wqu_ref, wki_ref, wqi_ref, wku_ref,
             qu_ref, ki_ref, qi_ref, ku_ref):
    xu = xu_ref[...]
    xi = xi_ref[...]
    qu_ref[...] = jnp.dot(xu, wqu_ref[...], preferred_element_type=jnp.float32)
    ki_ref[...] = jnp.dot(xi, wki_ref[...], preferred_element_type=jnp.float32)
    qi_ref[...] = jnp.dot(xi, wqi_ref[...], preferred_element_type=jnp.float32)
    ku_ref[...] = jnp.dot(xu, wku_ref[...], preferred_element_type=jnp.float32)


_QK_BLK = 2000


def _tc_qk(x_user, x_item, wqu, wki, wqi, wku):
    n_blk = N // _QK_BLK
    row_spec = pl.BlockSpec((_QK_BLK, D), lambda i: (i, 0))
    w_spec = pl.BlockSpec((D, D), lambda i: (0, 0))
    out = jax.ShapeDtypeStruct((N, D), jnp.float32)
    return pl.pallas_call(
        _qk_body,
        grid=(n_blk,),
        in_specs=[row_spec, row_spec, w_spec, w_spec, w_spec, w_spec],
        out_specs=[row_spec] * 4,
        out_shape=[out] * 4,
    )(x_user, x_item, wqu, wki, wqi, wku)


def _finish_body(u_ref, s_ref, wv_ref, o_ref):
    acc = jnp.zeros(o_ref.shape, jnp.float32)
    for h in range(H):
        u = u_ref[h]
        sh = s_ref[:, h:h + 1]
        zp = u / (sh + 1e-9)
        acc = acc + jax.nn.relu(
            jnp.dot(zp, wv_ref[h], preferred_element_type=jnp.float32))
    o_ref[...] = acc * (1.0 / H)


_FIN_BLK = 2000


def _tc_finish(u, s, wv):
    n_blk = N // _FIN_BLK
    return pl.pallas_call(
        _finish_body,
        grid=(n_blk,),
        in_specs=[
            pl.BlockSpec((H, _FIN_BLK, D), lambda i: (0, i, 0)),
            pl.BlockSpec((_FIN_BLK, H), lambda i: (i, 0)),
            pl.BlockSpec((H, D, D), lambda i: (0, 0, 0)),
        ],
        out_specs=pl.BlockSpec((_FIN_BLK, D), lambda i: (i, 0)),
        out_shape=jax.ShapeDtypeStruct((N, D), jnp.float32),
    )(u, s, wv)


def _edge_stages(q, k, src, dst):
    lh, mp = _pass_a1(q, k, src, dst)
    m = _combine_max(mp)
    eh, sp = _pass_a2(lh, m, dst)
    s = _combine_sum(sp)
    return eh, s.reshape(NPAD, H)


def _cat_heads(w):
    return jnp.transpose(w, (1, 0, 2)).reshape(D, H * DH)


@jax.jit
def kernel(u2i, i2u, x_user, x_item, user_w_q, user_w_k, user_w_v,
           item_w_q, item_w_k, item_w_v):
    wqu = _cat_heads(user_w_q)
    wki = _cat_heads(item_w_k)
    wqi = _cat_heads(item_w_q)
    wku = _cat_heads(user_w_k)
    qu, ki, qi, ku = _tc_qk(x_user, x_item, wqu, wki, wqi, wku)

    eh_u, s_u = _edge_stages(qu, ki, i2u[0], i2u[1])
    eh_i, s_i = _edge_stages(qi, ku, u2i[0], u2i[1])

    u_u, u_i = _pass_b(x_item, i2u[0], i2u[1], eh_u,
                       x_user, u2i[0], u2i[1], eh_i)

    z_user = _tc_finish(u_u, s_u, item_w_v)
    z_item = _tc_finish(u_i, s_i, user_w_v)
    return (z_user, z_item)


# speedup vs baseline: 3.3099x; 1.0336x over previous
import functools

import jax
import jax.numpy as jnp
from jax import lax
from jax.experimental import pallas as pl
from jax.experimental.pallas import tpu as pltpu
from jax.experimental.pallas import tpu_sc as plsc

N = 10000
E = 320000
D = 128
DH = 32
H = 4
NPAD = 10240
NW = 32
NT = 16
EW = E // NW
ES = E // NT
C = 128
UROWS = NPAD // NT
MFLAT = NPAD * H
MSL = MFLAT // NW
NEG = -1e30
A_TAIL = EW - (EW // C) * C
B_TAIL = ES - (ES // C) * C

_mesh = plsc.VectorSubcoreMesh(core_axis_name="c", subcore_axis_name="s")
_iota16 = lambda: lax.iota(jnp.int32, 16)
_sc_params = pltpu.CompilerParams(needs_layout_passes=False)


def _zero_rows(ref, nrows, ncols):
    z = jnp.zeros((16,), jnp.float32)

    def body(r, carry):
        for j in range(ncols // 16):
            ref[r, pl.ds(j * 16, 16)] = z
        return carry

    lax.fori_loop(0, nrows, body, None)


def _fill_flat(ref, n, value):
    v = jnp.full((16,), value, jnp.float32)

    def body(i, carry):
        ref[pl.ds(i * 16, 16)] = v
        return carry

    lax.fori_loop(0, n // 16, body, None)


@functools.partial(
    pl.kernel,
    out_type=(
        jax.ShapeDtypeStruct((H * E,), jnp.float32),
        jax.ShapeDtypeStruct((NW * MFLAT,), jnp.float32),
    ),
    mesh=_mesh,
    compiler_params=_sc_params,
    scratch_types=[
        pltpu.VMEM((C,), jnp.int32),
        pltpu.VMEM((C,), jnp.int32),
        pltpu.VMEM((C, D), jnp.float32),
        pltpu.VMEM((C, D), jnp.float32),
        pltpu.VMEM((H, C), jnp.float32),
        pltpu.VMEM((MFLAT,), jnp.float32),
        pltpu.SemaphoreType.DMA,
        pltpu.SemaphoreType.DMA,
    ],
)
def _pass_a1(q_hbm, k_hbm, src_hbm, dst_hbm, l_hbm, mp_hbm,
             idx_d, idx_s, qbuf, kbuf, lbuf, m_tile, sem0, sem1):
    cid = lax.axis_index("c")
    sid = lax.axis_index("s")
    wid = cid * NT + sid

    _fill_flat(m_tile, MFLAT, NEG)
    iota = _iota16()

    def do_chunk(base, csz):
        ng = csz // 16
        pltpu.sync_copy(dst_hbm.at[pl.ds(base, csz)], idx_d.at[pl.ds(0, csz)])
        pltpu.sync_copy(src_hbm.at[pl.ds(base, csz)], idx_s.at[pl.ds(0, csz)])
        cp0 = pltpu.async_copy(q_hbm.at[idx_d], qbuf, sem0)
        cp1 = pltpu.async_copy(k_hbm.at[idx_s], kbuf, sem1)
        cp0.wait()
        cp1.wait()

        def group(g, carry):
            rows = g * 16 + iota
            dstv = idx_d[pl.ds(g * 16, 16)]
            accs = [jnp.zeros((16,), jnp.float32) for _ in range(H)]
            for d in range(D):
                cols = jnp.full((16,), d, jnp.int32)
                qv = plsc.load_gather(qbuf, [rows, cols])
                kv = plsc.load_gather(kbuf, [rows, cols])
                accs[d // DH] = accs[d // DH] + qv * kv
            for h in range(H):
                plsc.store_scatter(lbuf, [jnp.full((16,), h, jnp.int32), rows],
                                   accs[h])
                flat = dstv * H + h
                old = plsc.load_gather(m_tile, [flat])
                v = jnp.maximum(old, accs[h])
                sv, sf = plsc.sort_key_val(v, flat, descending=False)
                unused_cnt, last = plsc.scan_count(sf)
                plsc.store_scatter(m_tile, [sf], sv, mask=last)
            return carry

        lax.fori_loop(0, ng, group, None)
        for h in range(H):
            pltpu.sync_copy(lbuf.at[h, pl.ds(0, csz)],
                            l_hbm.at[pl.ds(h * E + base, csz)])

    def chunk_body(i, carry):
        do_chunk(wid * EW + i * C, C)
        return carry

    lax.fori_loop(0, EW // C, chunk_body, None)
    do_chunk(wid * EW + (EW // C) * C, A_TAIL)

    pltpu.sync_copy(m_tile, mp_hbm.at[pl.ds(wid * MFLAT, MFLAT)])


def _make_combine(op):
    @functools.partial(
        pl.kernel,
        out_type=jax.ShapeDtypeStruct((MFLAT,), jnp.float32),
        mesh=_mesh,
        compiler_params=_sc_params,
        scratch_types=[
            pltpu.VMEM((MSL,), jnp.float32),
            pltpu.VMEM((MSL,), jnp.float32),
        ],
    )
    def combine(mp_hbm, out_hbm, acc, tmp):
        cid = lax.axis_index("c")
        sid = lax.axis_index("s")
        wid = cid * NT + sid
        off = wid * MSL

        pltpu.sync_copy(mp_hbm.at[pl.ds(off, MSL)], acc)
        for t in range(1, NW):
            pltpu.sync_copy(mp_hbm.at[pl.ds(t * MFLAT + off, MSL)], tmp)

            def body(i, carry):
                sl = pl.ds(i * 16, 16)
                acc[sl] = op(acc[sl], tmp[sl])
                return carry

            lax.fori_loop(0, MSL // 16, body, None)
        pltpu.sync_copy(acc, out_hbm.at[pl.ds(off, MSL)])

    return combine


_combine_max = _make_combine(jnp.maximum)
_combine_sum = _make_combine(lambda a, b: a + b)


@functools.partial(
    pl.kernel,
    out_type=(
        jax.ShapeDtypeStruct((H * E,), jnp.float32),
        jax.ShapeDtypeStruct((NW * MFLAT,), jnp.float32),
    ),
    mesh=_mesh,
    compiler_params=_sc_params,
    scratch_types=[
        pltpu.VMEM((C,), jnp.int32),
        pltpu.VMEM((H, C), jnp.float32),
        pltpu.VMEM((H, C), jnp.float32),
        pltpu.VMEM((MFLAT,), jnp.float32),
        pltpu.VMEM((MFLAT,), jnp.float32),
    ],
)
def _pass_a2(l_hbm, m_hbm, dst_hbm, e_hbm, sp_hbm,
             idx_d, lbuf, ebuf, m0, s_tile):
    cid = lax.axis_index("c")
    sid = lax.axis_index("s")
    wid = cid * NT + sid

    pltpu.sync_copy(m_hbm, m0)
    _fill_flat(s_tile, MFLAT, 0.0)

    iota = _iota16()

    def accumulate(flat, ev):
        def cond(mask):
            return jnp.any(mask)

        def body(mask):
            unused_cnt, last = plsc.scan_count(flat, mask)
            commit = jnp.logical_and(mask, last)
            old = plsc.load_gather(s_tile, [flat])
            plsc.store_scatter(s_tile, [flat], old + ev, mask=commit)
            return jnp.logical_and(mask, jnp.logical_not(commit))

        lax.while_loop(cond, body, jnp.full((16,), True, jnp.bool_))

    def do_chunk(base, csz):
        ng = csz // 16
        pltpu.sync_copy(dst_hbm.at[pl.ds(base, csz)], idx_d.at[pl.ds(0, csz)])
        for h in range(H):
            pltpu.sync_copy(l_hbm.at[pl.ds(h * E + base, csz)],
                            lbuf.at[h, pl.ds(0, csz)])

        def group(g, carry):
            rows = g * 16 + iota
            dstv = idx_d[pl.ds(g * 16, 16)]
            for h in range(H):
                hv = jnp.full((16,), h, jnp.int32)
                lv = plsc.load_gather(lbuf, [hv, rows])
                flat = dstv * H + h
                mv = plsc.load_gather(m0, [flat])
                ev = jnp.exp(lv - mv)
                plsc.store_scatter(ebuf, [hv, rows], ev)
                accumulate(flat, ev)
            return carry

        lax.fori_loop(0, ng, group, None)
        for h in range(H):
            pltpu.sync_copy(ebuf.at[h, pl.ds(0, csz)],
                            e_hbm.at[pl.ds(h * E + base, csz)])

    def chunk_body(i, carry):
        do_chunk(wid * EW + i * C, C)
        return carry

    lax.fori_loop(0, EW // C, chunk_body, None)
    do_chunk(wid * EW + (EW // C) * C, A_TAIL)

    pltpu.sync_copy(s_tile, sp_hbm.at[pl.ds(wid * MFLAT, MFLAT)])


BP = 256
NPAIR = ES // BP
B_TAIL2 = ES - NPAIR * BP


@functools.partial(
    pl.kernel,
    out_type=(
        jax.ShapeDtypeStruct((H, NPAD, D), jnp.float32),
        jax.ShapeDtypeStruct((H, NPAD, D), jnp.float32),
    ),
    mesh=_mesh,
    compiler_params=_sc_params,
    scratch_types=[
        pltpu.VMEM((BP,), jnp.int32),
        pltpu.VMEM((BP,), jnp.int32),
        pltpu.VMEM((BP,), jnp.float32),
        pltpu.VMEM((C,), jnp.int32),
        pltpu.VMEM((C,), jnp.int32),
        pltpu.VMEM((C,), jnp.int32),
        pltpu.VMEM((C,), jnp.int32),
        pltpu.VMEM((B_TAIL,), jnp.int32),
        pltpu.VMEM((C, D), jnp.float32),
        pltpu.VMEM((C, D), jnp.float32),
        pltpu.VMEM_SHARED((NPAD, D), jnp.float32),
        pltpu.SemaphoreType.DMA,
        pltpu.SemaphoreType.DMA,
    ],
)
def _pass_b(x0_hbm, src0_hbm, dst0_hbm, e0_hbm,
            x1_hbm, src1_hbm, dst1_hbm, e1_hbm, u0_hbm, u1_hbm,
            sbig, dbig, ebig, idxw0, idxw1, idxg0, idxg1, idxwt,
            xbuf0, xbuf1, u_shared, gsem0, gsem1):
    cid = lax.axis_index("c")
    sid = lax.axis_index("s")
    iota = _iota16()
    zi = jnp.zeros((16,), jnp.int32)

    def compute_chunk(xbuf, idxw, off):
        def group(g, carry):
            rows = g * 16 + iota
            wv = ebig[pl.ds(off + g * 16, 16)]
            idxw[pl.ds(g * 16, 16)] = dbig[pl.ds(off + g * 16, 16)]
            for d in range(D):
                cols = jnp.full((16,), d, jnp.int32)
                xv = plsc.load_gather(xbuf, [rows, cols])
                plsc.store_scatter(xbuf, [rows, cols], xv * wv)
            return carry

        lax.fori_loop(0, 8, group, None)

    for x_hbm, src_hbm, dst_hbm, e_hbm, u_hbm in (
            (x0_hbm, src0_hbm, dst0_hbm, e0_hbm, u0_hbm),
            (x1_hbm, src1_hbm, dst1_hbm, e1_hbm, u1_hbm)):
        for ph in range(2):
            h = 2 * ph + cid
            _zero_rows(xbuf0, C, D)
            for z in range(UROWS // C):
                pltpu.sync_copy(xbuf0,
                                u_shared.at[pl.ds(sid * UROWS + z * C, C)])
            plsc.subcore_barrier()

            def pair_body(i, carry, src_hbm=src_hbm, dst_hbm=dst_hbm,
                          e_hbm=e_hbm, x_hbm=x_hbm, h=h):
                base = sid * ES + i * BP
                pltpu.sync_copy(src_hbm.at[pl.ds(base, BP)], sbig)
                pltpu.sync_copy(dst_hbm.at[pl.ds(base, BP)], dbig)
                pltpu.sync_copy(e_hbm.at[pl.ds(h * E + base, BP)], ebig)

                def cpidx(k, carry):
                    idxg0[pl.ds(k * 16, 16)] = sbig[pl.ds(k * 16, 16)]
                    idxg1[pl.ds(k * 16, 16)] = sbig[pl.ds(C + k * 16, 16)]
                    return carry

                lax.fori_loop(0, C // 16, cpidx, None)
                g0 = pltpu.async_copy(x_hbm.at[idxg0], xbuf0, gsem0)
                g1 = pltpu.async_copy(x_hbm.at[idxg1], xbuf1, gsem1)
                g0.wait()
                compute_chunk(xbuf0, idxw0, 0)
                pltpu.sync_copy(xbuf0, u_shared.at[idxw0], add=True)
                g1.wait()
                compute_chunk(xbuf1, idxw1, C)
                pltpu.sync_copy(xbuf1, u_shared.at[idxw1], add=True)
                return carry

            lax.fori_loop(0, NPAIR, pair_body, None)

            tbase = sid * ES + NPAIR * BP
            pltpu.sync_copy(src_hbm.at[pl.ds(tbase, B_TAIL2)],
                            sbig.at[pl.ds(0, B_TAIL2)])
            pltpu.sync_copy(dst_hbm.at[pl.ds(tbase, B_TAIL2)], idxwt)
            pltpu.sync_copy(e_hbm.at[pl.ds(h * E + tbase, B_TAIL2)],
                            ebig.at[pl.ds(0, B_TAIL2)])
            for k in range(B_TAIL2 // 16):
                idxg0[pl.ds(k * 16, 16)] = sbig[pl.ds(k * 16, 16)]
            pltpu.async_copy(x_hbm.at[idxg0.at[pl.ds(0, B_TAIL2)]],
                             xbuf0.at[pl.ds(0, B_TAIL2)], gsem0).wait()

            def tail_group(g, carry):
                rows = g * 16 + iota
                wv = ebig[pl.ds(g * 16, 16)]
                for d in range(D):
                    cols = jnp.full((16,), d, jnp.int32)
                    xv = plsc.load_gather(xbuf0, [rows, cols])
                    plsc.store_scatter(xbuf0, [rows, cols], xv * wv)
                return carry

            lax.fori_loop(0, B_TAIL2 // 16, tail_group, None)
            pltpu.sync_copy(xbuf0.at[pl.ds(0, B_TAIL2)], u_shared.at[idxwt],
                            add=True)

            plsc.subcore_barrier()
            for z in range(UROWS // C):
                r = sid * UROWS + z * C
                pltpu.sync_copy(u_shared.at[pl.ds(r, C)],
                                u_hbm.at[h, pl.ds(r, C)])
            plsc.subcore_barrier()


def _qk_body(xu_ref, xi_ref, wqu_ref, wki_ref, wqi_ref, wku_ref,
             qu_ref, ki_ref, qi_ref, ku_ref):
    xu = xu_ref[...]
    xi = xi_ref[...]
    qu_ref[...] = jnp.dot(xu, wqu_ref[...], preferred_element_type=jnp.float32)
    ki_ref[...] = jnp.dot(xi, wki_ref[...], preferred_element_type=jnp.float32)
    qi_ref[...] = jnp.dot(xi, wqi_ref[...], preferred_element_type=jnp.float32)
    ku_ref[...] = jnp.dot(xu, wku_ref[...], preferred_element_type=jnp.float32)


_QK_BLK = 2000


def _tc_qk(x_user, x_item, wqu, wki, wqi, wku):
    n_blk = N // _QK_BLK
    row_spec = pl.BlockSpec((_QK_BLK, D), lambda i: (i, 0))
    w_spec = pl.BlockSpec((D, D), lambda i: (0, 0))
    out = jax.ShapeDtypeStruct((N, D), jnp.float32)
    return pl.pallas_call(
        _qk_body,
        grid=(n_blk,),
        in_specs=[row_spec, row_spec, w_spec, w_spec, w_spec, w_spec],
        out_specs=[row_spec] * 4,
        out_shape=[out] * 4,
    )(x_user, x_item, wqu, wki, wqi, wku)


def _finish_body(u_ref, s_ref, wv_ref, o_ref):
    acc = jnp.zeros(o_ref.shape, jnp.float32)
    for h in range(H):
        u = u_ref[h]
        sh = s_ref[:, h:h + 1]
        zp = u / (sh + 1e-9)
        acc = acc + jax.nn.relu(
            jnp.dot(zp, wv_ref[h], preferred_element_type=jnp.float32))
    o_ref[...] = acc * (1.0 / H)


_FIN_BLK = 2000


def _tc_finish(u, s, wv):
    n_blk = N // _FIN_BLK
    return pl.pallas_call(
        _finish_body,
        grid=(n_blk,),
        in_specs=[
            pl.BlockSpec((H, _FIN_BLK, D), lambda i: (0, i, 0)),
            pl.BlockSpec((_FIN_BLK, H), lambda i: (i, 0)),
            pl.BlockSpec((H, D, D), lambda i: (0, 0, 0)),
        ],
        out_specs=pl.BlockSpec((_FIN_BLK, D), lambda i: (i, 0)),
        out_shape=jax.ShapeDtypeStruct((N, D), jnp.float32),
    )(u, s, wv)


def _edge_stages(q, k, src, dst):
    lh, mp = _pass_a1(q, k, src, dst)
    m = _combine_max(mp)
    eh, sp = _pass_a2(lh, m, dst)
    s = _combine_sum(sp)
    return eh, s.reshape(NPAD, H)


def _cat_heads(w):
    return jnp.transpose(w, (1, 0, 2)).reshape(D, H * DH)


@jax.jit
def kernel(u2i, i2u, x_user, x_item, user_w_q, user_w_k, user_w_v,
           item_w_q, item_w_k, item_w_v):
    wqu = _cat_heads(user_w_q)
    wki = _cat_heads(item_w_k)
    wqi = _cat_heads(item_w_q)
    wku = _cat_heads(user_w_k)
    qu, ki, qi, ku = _tc_qk(x_user, x_item, wqu, wki, wqi, wku)

    eh_u, s_u = _edge_stages(qu, ki, i2u[0], i2u[1])
    eh_i, s_i = _edge_stages(qi, ku, u2i[0], u2i[1])

    u_u, u_i = _pass_b(x_item, i2u[0], i2u[1], eh_u,
                       x_user, u2i[0], u2i[1], eh_i)

    z_user = _tc_finish(u_u, s_u, item_w_v)
    z_item = _tc_finish(u_i, s_i, user_w_v)
    return (z_user, z_item)


# speedup vs baseline: 10.0864x; 3.0473x over previous
import functools

import jax
import jax.numpy as jnp
from jax import lax
from jax.experimental import pallas as pl
from jax.experimental.pallas import tpu as pltpu
from jax.experimental.pallas import tpu_sc as plsc

N = 10000
E = 320000
D = 128
DH = 32
H = 4
NPAD = 10240
NW = 32
NT = 16
EW = E // NW
ES = E // NT
C = 128
UROWS = NPAD // NT
MFLAT = NPAD * H
MSL = MFLAT // NW
NEG = -1e30
A_TAIL = EW - (EW // C) * C
B_TAIL = ES - (ES // C) * C

_mesh = plsc.VectorSubcoreMesh(core_axis_name="c", subcore_axis_name="s")
_iota16 = lambda: lax.iota(jnp.int32, 16)
_sc_params = pltpu.CompilerParams(needs_layout_passes=False)


def _zero_rows(ref, nrows, ncols):
    z = jnp.zeros((16,), jnp.float32)

    def body(r, carry):
        for j in range(ncols // 16):
            ref[r, pl.ds(j * 16, 16)] = z
        return carry

    lax.fori_loop(0, nrows, body, None)


def _fill_flat(ref, n, value):
    v = jnp.full((16,), value, jnp.float32)

    def body(i, carry):
        ref[pl.ds(i * 16, 16)] = v
        return carry

    lax.fori_loop(0, n // 16, body, None)


@functools.partial(
    pl.kernel,
    out_type=(
        jax.ShapeDtypeStruct((H * E,), jnp.float32),
        jax.ShapeDtypeStruct((NW * MFLAT,), jnp.float32),
    ),
    mesh=_mesh,
    compiler_params=_sc_params,
    scratch_types=[
        pltpu.VMEM((C,), jnp.int32),
        pltpu.VMEM((C,), jnp.int32),
        pltpu.VMEM((C, D), jnp.float32),
        pltpu.VMEM((C, D), jnp.float32),
        pltpu.VMEM((H, C), jnp.float32),
        pltpu.VMEM((MFLAT,), jnp.float32),
        pltpu.SemaphoreType.DMA,
        pltpu.SemaphoreType.DMA,
    ],
)
def _pass_a1(q_hbm, k_hbm, src_hbm, dst_hbm, l_hbm, mp_hbm,
             idx_d, idx_s, qbuf, kbuf, lbuf, m_tile, sem0, sem1):
    cid = lax.axis_index("c")
    sid = lax.axis_index("s")
    wid = cid * NT + sid

    _fill_flat(m_tile, MFLAT, NEG)
    iota = _iota16()

    def do_chunk(base, csz):
        ng = csz // 16
        pltpu.sync_copy(dst_hbm.at[pl.ds(base, csz)], idx_d.at[pl.ds(0, csz)])
        pltpu.sync_copy(src_hbm.at[pl.ds(base, csz)], idx_s.at[pl.ds(0, csz)])
        cp0 = pltpu.async_copy(q_hbm.at[idx_d], qbuf, sem0)
        cp1 = pltpu.async_copy(k_hbm.at[idx_s], kbuf, sem1)
        cp0.wait()
        cp1.wait()

        def group(g, carry):
            rows = g * 16 + iota
            dstv = idx_d[pl.ds(g * 16, 16)]
            accs = [jnp.zeros((16,), jnp.float32) for _ in range(H)]
            for d in range(D):
                cols = jnp.full((16,), d, jnp.int32)
                qv = plsc.load_gather(qbuf, [rows, cols])
                kv = plsc.load_gather(kbuf, [rows, cols])
                accs[d // DH] = accs[d // DH] + qv * kv
            for h in range(H):
                plsc.store_scatter(lbuf, [jnp.full((16,), h, jnp.int32), rows],
                                   accs[h])
                flat = dstv * H + h
                old = plsc.load_gather(m_tile, [flat])
                v = jnp.maximum(old, accs[h])
                sv, sf = plsc.sort_key_val(v, flat, descending=False)
                unused_cnt, last = plsc.scan_count(sf)
                plsc.store_scatter(m_tile, [sf], sv, mask=last)
            return carry

        lax.fori_loop(0, ng, group, None)
        for h in range(H):
            pltpu.sync_copy(lbuf.at[h, pl.ds(0, csz)],
                            l_hbm.at[pl.ds(h * E + base, csz)])

    def chunk_body(i, carry):
        do_chunk(wid * EW + i * C, C)
        return carry

    lax.fori_loop(0, EW // C, chunk_body, None)
    do_chunk(wid * EW + (EW // C) * C, A_TAIL)

    pltpu.sync_copy(m_tile, mp_hbm.at[pl.ds(wid * MFLAT, MFLAT)])


def _make_combine(op):
    @functools.partial(
        pl.kernel,
        out_type=jax.ShapeDtypeStruct((MFLAT,), jnp.float32),
        mesh=_mesh,
        compiler_params=_sc_params,
        scratch_types=[
            pltpu.VMEM((MSL,), jnp.float32),
            pltpu.VMEM((MSL,), jnp.float32),
        ],
    )
    def combine(mp_hbm, out_hbm, acc, tmp):
        cid = lax.axis_index("c")
        sid = lax.axis_index("s")
        wid = cid * NT + sid
        off = wid * MSL

        pltpu.sync_copy(mp_hbm.at[pl.ds(off, MSL)], acc)
        for t in range(1, NW):
            pltpu.sync_copy(mp_hbm.at[pl.ds(t * MFLAT + off, MSL)], tmp)

            def body(i, carry):
                sl = pl.ds(i * 16, 16)
                acc[sl] = op(acc[sl], tmp[sl])
                return carry

            lax.fori_loop(0, MSL // 16, body, None)
        pltpu.sync_copy(acc, out_hbm.at[pl.ds(off, MSL)])

    return combine


_combine_max = _make_combine(jnp.maximum)
_combine_sum = _make_combine(lambda a, b: a + b)


@functools.partial(
    pl.kernel,
    out_type=(
        jax.ShapeDtypeStruct((H * E,), jnp.float32),
        jax.ShapeDtypeStruct((NW * MFLAT,), jnp.float32),
    ),
    mesh=_mesh,
    compiler_params=_sc_params,
    scratch_types=[
        pltpu.VMEM((C,), jnp.int32),
        pltpu.VMEM((H, C), jnp.float32),
        pltpu.VMEM((H, C), jnp.float32),
        pltpu.VMEM((MFLAT,), jnp.float32),
        pltpu.VMEM((MFLAT,), jnp.float32),
    ],
)
def _pass_a2(l_hbm, m_hbm, dst_hbm, e_hbm, sp_hbm,
             idx_d, lbuf, ebuf, m0, s_tile):
    cid = lax.axis_index("c")
    sid = lax.axis_index("s")
    wid = cid * NT + sid

    pltpu.sync_copy(m_hbm, m0)
    _fill_flat(s_tile, MFLAT, 0.0)

    iota = _iota16()

    def accumulate(flat, ev):
        def cond(mask):
            return jnp.any(mask)

        def body(mask):
            unused_cnt, last = plsc.scan_count(flat, mask)
            commit = jnp.logical_and(mask, last)
            old = plsc.load_gather(s_tile, [flat])
            plsc.store_scatter(s_tile, [flat], old + ev, mask=commit)
            return jnp.logical_and(mask, jnp.logical_not(commit))

        lax.while_loop(cond, body, jnp.full((16,), True, jnp.bool_))

    def do_chunk(base, csz):
        ng = csz // 16
        pltpu.sync_copy(dst_hbm.at[pl.ds(base, csz)], idx_d.at[pl.ds(0, csz)])
        for h in range(H):
            pltpu.sync_copy(l_hbm.at[pl.ds(h * E + base, csz)],
                            lbuf.at[h, pl.ds(0, csz)])

        def group(g, carry):
            rows = g * 16 + iota
            dstv = idx_d[pl.ds(g * 16, 16)]
            for h in range(H):
                hv = jnp.full((16,), h, jnp.int32)
                lv = plsc.load_gather(lbuf, [hv, rows])
                flat = dstv * H + h
                mv = plsc.load_gather(m0, [flat])
                ev = jnp.exp(lv - mv)
                plsc.store_scatter(ebuf, [hv, rows], ev)
                accumulate(flat, ev)
            return carry

        lax.fori_loop(0, ng, group, None)
        for h in range(H):
            pltpu.sync_copy(ebuf.at[h, pl.ds(0, csz)],
                            e_hbm.at[pl.ds(h * E + base, csz)])

    def chunk_body(i, carry):
        do_chunk(wid * EW + i * C, C)
        return carry

    lax.fori_loop(0, EW // C, chunk_body, None)
    do_chunk(wid * EW + (EW // C) * C, A_TAIL)

    pltpu.sync_copy(s_tile, sp_hbm.at[pl.ds(wid * MFLAT, MFLAT)])


BP = 256
NPAIR = ES // BP
B_TAIL2 = ES - NPAIR * BP


@functools.partial(
    pl.kernel,
    out_type=(
        jax.ShapeDtypeStruct((H, NPAD, D), jnp.float32),
        jax.ShapeDtypeStruct((H, NPAD, D), jnp.float32),
    ),
    mesh=_mesh,
    compiler_params=_sc_params,
    scratch_types=[
        pltpu.VMEM((BP,), jnp.int32),
        pltpu.VMEM((BP,), jnp.int32),
        pltpu.VMEM((BP,), jnp.float32),
        pltpu.VMEM((C,), jnp.int32),
        pltpu.VMEM((C,), jnp.int32),
        pltpu.VMEM((C,), jnp.int32),
        pltpu.VMEM((C,), jnp.int32),
        pltpu.VMEM((B_TAIL,), jnp.int32),
        pltpu.VMEM((C, D), jnp.float32),
        pltpu.VMEM((C, D), jnp.float32),
        pltpu.VMEM_SHARED((NPAD, D), jnp.float32),
        pltpu.SemaphoreType.DMA,
        pltpu.SemaphoreType.DMA,
    ],
)
def _pass_b(x0_hbm, src0_hbm, dst0_hbm, e0_hbm,
            x1_hbm, src1_hbm, dst1_hbm, e1_hbm, u0_hbm, u1_hbm,
            sbig, dbig, ebig, idxw0, idxw1, idxg0, idxg1, idxwt,
            xbuf0, xbuf1, u_shared, gsem0, gsem1):
    cid = lax.axis_index("c")
    sid = lax.axis_index("s")
    iota = _iota16()
    zi = jnp.zeros((16,), jnp.int32)

    def compute_chunk(xbuf, idxw, off):
        def cpidx(k, carry):
            idxw[pl.ds(k * 16, 16)] = dbig[pl.ds(off + k * 16, 16)]
            return carry

        lax.fori_loop(0, C // 16, cpidx, None)

        def group(g, carry):
            wv = ebig[pl.ds(off + g * 16, 16)]
            for lane in range(16):
                w = wv[lane]
                r = g * 16 + lane
                for j in range(D // 16):
                    sl = pl.ds(j * 16, 16)
                    xbuf[r, sl] = xbuf[r, sl] * w
            return carry

        lax.fori_loop(0, C // 16, group, None)

    for x_hbm, src_hbm, dst_hbm, e_hbm, u_hbm in (
            (x0_hbm, src0_hbm, dst0_hbm, e0_hbm, u0_hbm),
            (x1_hbm, src1_hbm, dst1_hbm, e1_hbm, u1_hbm)):
        for ph in range(2):
            h = 2 * ph + cid
            _zero_rows(xbuf0, C, D)
            for z in range(UROWS // C):
                pltpu.sync_copy(xbuf0,
                                u_shared.at[pl.ds(sid * UROWS + z * C, C)])
            plsc.subcore_barrier()

            def pair_body(i, carry, src_hbm=src_hbm, dst_hbm=dst_hbm,
                          e_hbm=e_hbm, x_hbm=x_hbm, h=h):
                base = sid * ES + i * BP
                pltpu.sync_copy(src_hbm.at[pl.ds(base, BP)], sbig)
                pltpu.sync_copy(dst_hbm.at[pl.ds(base, BP)], dbig)
                pltpu.sync_copy(e_hbm.at[pl.ds(h * E + base, BP)], ebig)

                def cpidx(k, carry):
                    idxg0[pl.ds(k * 16, 16)] = sbig[pl.ds(k * 16, 16)]
                    idxg1[pl.ds(k * 16, 16)] = sbig[pl.ds(C + k * 16, 16)]
                    return carry

                lax.fori_loop(0, C // 16, cpidx, None)
                g0 = pltpu.async_copy(x_hbm.at[idxg0], xbuf0, gsem0)
                g1 = pltpu.async_copy(x_hbm.at[idxg1], xbuf1, gsem1)
                g0.wait()
                compute_chunk(xbuf0, idxw0, 0)
                pltpu.sync_copy(xbuf0, u_shared.at[idxw0], add=True)
                g1.wait()
                compute_chunk(xbuf1, idxw1, C)
                pltpu.sync_copy(xbuf1, u_shared.at[idxw1], add=True)
                return carry

            lax.fori_loop(0, NPAIR, pair_body, None)

            tbase = sid * ES + NPAIR * BP
            pltpu.sync_copy(src_hbm.at[pl.ds(tbase, B_TAIL2)],
                            sbig.at[pl.ds(0, B_TAIL2)])
            pltpu.sync_copy(dst_hbm.at[pl.ds(tbase, B_TAIL2)], idxwt)
            pltpu.sync_copy(e_hbm.at[pl.ds(h * E + tbase, B_TAIL2)],
                            ebig.at[pl.ds(0, B_TAIL2)])
            for k in range(B_TAIL2 // 16):
                idxg0[pl.ds(k * 16, 16)] = sbig[pl.ds(k * 16, 16)]
            pltpu.async_copy(x_hbm.at[idxg0.at[pl.ds(0, B_TAIL2)]],
                             xbuf0.at[pl.ds(0, B_TAIL2)], gsem0).wait()

            def tail_group(g, carry):
                wv = ebig[pl.ds(g * 16, 16)]
                for lane in range(16):
                    w = wv[lane]
                    r = g * 16 + lane
                    for j in range(D // 16):
                        sl = pl.ds(j * 16, 16)
                        xbuf0[r, sl] = xbuf0[r, sl] * w
                return carry

            lax.fori_loop(0, B_TAIL2 // 16, tail_group, None)
            pltpu.sync_copy(xbuf0.at[pl.ds(0, B_TAIL2)], u_shared.at[idxwt],
                            add=True)

            plsc.subcore_barrier()
            for z in range(UROWS // C):
                r = sid * UROWS + z * C
                pltpu.sync_copy(u_shared.at[pl.ds(r, C)],
                                u_hbm.at[h, pl.ds(r, C)])
            plsc.subcore_barrier()


def _qk_body(xu_ref, xi_ref, wqu_ref, wki_ref, wqi_ref, wku_ref,
             qu_ref, ki_ref, qi_ref, ku_ref):
    xu = xu_ref[...]
    xi = xi_ref[...]
    qu_ref[...] = jnp.dot(xu, wqu_ref[...], preferred_element_type=jnp.float32)
    ki_ref[...] = jnp.dot(xi, wki_ref[...], preferred_element_type=jnp.float32)
    qi_ref[...] = jnp.dot(xi, wqi_ref[...], preferred_element_type=jnp.float32)
    ku_ref[...] = jnp.dot(xu, wku_ref[...], preferred_element_type=jnp.float32)


_QK_BLK = 2000


def _tc_qk(x_user, x_item, wqu, wki, wqi, wku):
    n_blk = N // _QK_BLK
    row_spec = pl.BlockSpec((_QK_BLK, D), lambda i: (i, 0))
    w_spec = pl.BlockSpec((D, D), lambda i: (0, 0))
    out = jax.ShapeDtypeStruct((N, D), jnp.float32)
    return pl.pallas_call(
        _qk_body,
        grid=(n_blk,),
        in_specs=[row_spec, row_spec, w_spec, w_spec, w_spec, w_spec],
        out_specs=[row_spec] * 4,
        out_shape=[out] * 4,
    )(x_user, x_item, wqu, wki, wqi, wku)


def _finish_body(u_ref, s_ref, wv_ref, o_ref):
    acc = jnp.zeros(o_ref.shape, jnp.float32)
    for h in range(H):
        u = u_ref[h]
        sh = s_ref[:, h:h + 1]
        zp = u / (sh + 1e-9)
        acc = acc + jax.nn.relu(
            jnp.dot(zp, wv_ref[h], preferred_element_type=jnp.float32))
    o_ref[...] = acc * (1.0 / H)


_FIN_BLK = 2000


def _tc_finish(u, s, wv):
    n_blk = N // _FIN_BLK
    return pl.pallas_call(
        _finish_body,
        grid=(n_blk,),
        in_specs=[
            pl.BlockSpec((H, _FIN_BLK, D), lambda i: (0, i, 0)),
            pl.BlockSpec((_FIN_BLK, H), lambda i: (i, 0)),
            pl.BlockSpec((H, D, D), lambda i: (0, 0, 0)),
        ],
        out_specs=pl.BlockSpec((_FIN_BLK, D), lambda i: (i, 0)),
        out_shape=jax.ShapeDtypeStruct((N, D), jnp.float32),
    )(u, s, wv)


def _edge_stages(q, k, src, dst):
    lh, mp = _pass_a1(q, k, src, dst)
    m = _combine_max(mp)
    eh, sp = _pass_a2(lh, m, dst)
    s = _combine_sum(sp)
    return eh, s.reshape(NPAD, H)


def _cat_heads(w):
    return jnp.transpose(w, (1, 0, 2)).reshape(D, H * DH)


@jax.jit
def kernel(u2i, i2u, x_user, x_item, user_w_q, user_w_k, user_w_v,
           item_w_q, item_w_k, item_w_v):
    wqu = _cat_heads(user_w_q)
    wki = _cat_heads(item_w_k)
    wqi = _cat_heads(item_w_q)
    wku = _cat_heads(user_w_k)
    qu, ki, qi, ku = _tc_qk(x_user, x_item, wqu, wki, wqi, wku)

    eh_u, s_u = _edge_stages(qu, ki, i2u[0], i2u[1])
    eh_i, s_i = _edge_stages(qi, ku, u2i[0], u2i[1])

    u_u, u_i = _pass_b(x_item, i2u[0], i2u[1], eh_u,
                       x_user, u2i[0], u2i[1], eh_i)

    z_user = _tc_finish(u_u, s_u, item_w_v)
    z_item = _tc_finish(u_i, s_i, user_w_v)
    return (z_user, z_item)


# speedup vs baseline: 14.7961x; 1.4669x over previous
import functools

import jax
import jax.numpy as jnp
from jax import lax
from jax.experimental import pallas as pl
from jax.experimental.pallas import tpu as pltpu
from jax.experimental.pallas import tpu_sc as plsc

N = 10000
E = 320000
D = 128
DH = 32
H = 4
NPAD = 10240
NW = 32
NT = 16
EW = E // NW
ES = E // NT
C = 128
UROWS = NPAD // NT
MFLAT = NPAD * H
MSL = MFLAT // NW
NEG = -1e30
A_TAIL = EW - (EW // C) * C
B_TAIL = ES - (ES // C) * C

_mesh = plsc.VectorSubcoreMesh(core_axis_name="c", subcore_axis_name="s")
_iota16 = lambda: lax.iota(jnp.int32, 16)
_sc_params = pltpu.CompilerParams(needs_layout_passes=False)


def _zero_rows(ref, nrows, ncols):
    z = jnp.zeros((16,), jnp.float32)

    def body(r, carry):
        for j in range(ncols // 16):
            ref[r, pl.ds(j * 16, 16)] = z
        return carry

    lax.fori_loop(0, nrows, body, None)


def _fill_flat(ref, n, value):
    v = jnp.full((16,), value, jnp.float32)

    def body(i, carry):
        ref[pl.ds(i * 16, 16)] = v
        return carry

    lax.fori_loop(0, n // 16, body, None)


@functools.partial(
    pl.kernel,
    out_type=(
        jax.ShapeDtypeStruct((H * E,), jnp.float32),
        jax.ShapeDtypeStruct((NW * MFLAT,), jnp.float32),
    ),
    mesh=_mesh,
    compiler_params=_sc_params,
    scratch_types=[
        pltpu.VMEM((C,), jnp.int32),
        pltpu.VMEM((C,), jnp.int32),
        pltpu.VMEM((C, D), jnp.float32),
        pltpu.VMEM((C, D), jnp.float32),
        pltpu.VMEM((H, C), jnp.float32),
        pltpu.VMEM((MFLAT,), jnp.float32),
        pltpu.SemaphoreType.DMA,
        pltpu.SemaphoreType.DMA,
    ],
)
def _pass_a1(q_hbm, k_hbm, src_hbm, dst_hbm, l_hbm, mp_hbm,
             idx_d, idx_s, qbuf, kbuf, lbuf, m_tile, sem0, sem1):
    cid = lax.axis_index("c")
    sid = lax.axis_index("s")
    wid = cid * NT + sid

    _fill_flat(m_tile, MFLAT, NEG)
    iota = _iota16()

    def do_chunk(base, csz):
        ng = csz // 16
        pltpu.sync_copy(dst_hbm.at[pl.ds(base, csz)], idx_d.at[pl.ds(0, csz)])
        pltpu.sync_copy(src_hbm.at[pl.ds(base, csz)], idx_s.at[pl.ds(0, csz)])
        cp0 = pltpu.async_copy(q_hbm.at[idx_d], qbuf, sem0)
        cp1 = pltpu.async_copy(k_hbm.at[idx_s], kbuf, sem1)
        cp0.wait()
        cp1.wait()

        def group(g, carry):
            rows = g * 16 + iota
            dstv = idx_d[pl.ds(g * 16, 16)]
            accs = [jnp.zeros((16,), jnp.float32) for _ in range(H)]
            for lane in range(16):
                r = g * 16 + lane
                lm = iota == lane
                ps = []
                for j in range(D // 16):
                    sl = pl.ds(j * 16, 16)
                    ps.append(qbuf[r, sl] * kbuf[r, sl])
                for h in range(H):
                    t = ps[2 * h] + ps[2 * h + 1]
                    sh = jnp.sum(t)
                    accs[h] = jnp.where(lm, sh, accs[h])
            for h in range(H):
                plsc.store_scatter(lbuf, [jnp.full((16,), h, jnp.int32), rows],
                                   accs[h])
                flat = dstv * H + h
                old = plsc.load_gather(m_tile, [flat])
                v = jnp.maximum(old, accs[h])
                sv, sf = plsc.sort_key_val(v, flat, descending=False)
                unused_cnt, last = plsc.scan_count(sf)
                plsc.store_scatter(m_tile, [sf], sv, mask=last)
            return carry

        lax.fori_loop(0, ng, group, None)
        for h in range(H):
            pltpu.sync_copy(lbuf.at[h, pl.ds(0, csz)],
                            l_hbm.at[pl.ds(h * E + base, csz)])

    def chunk_body(i, carry):
        do_chunk(wid * EW + i * C, C)
        return carry

    lax.fori_loop(0, EW // C, chunk_body, None)
    do_chunk(wid * EW + (EW // C) * C, A_TAIL)

    pltpu.sync_copy(m_tile, mp_hbm.at[pl.ds(wid * MFLAT, MFLAT)])


def _make_combine(op):
    @functools.partial(
        pl.kernel,
        out_type=jax.ShapeDtypeStruct((MFLAT,), jnp.float32),
        mesh=_mesh,
        compiler_params=_sc_params,
        scratch_types=[
            pltpu.VMEM((MSL,), jnp.float32),
            pltpu.VMEM((MSL,), jnp.float32),
        ],
    )
    def combine(mp_hbm, out_hbm, acc, tmp):
        cid = lax.axis_index("c")
        sid = lax.axis_index("s")
        wid = cid * NT + sid
        off = wid * MSL

        pltpu.sync_copy(mp_hbm.at[pl.ds(off, MSL)], acc)
        for t in range(1, NW):
            pltpu.sync_copy(mp_hbm.at[pl.ds(t * MFLAT + off, MSL)], tmp)

            def body(i, carry):
                sl = pl.ds(i * 16, 16)
                acc[sl] = op(acc[sl], tmp[sl])
                return carry

            lax.fori_loop(0, MSL // 16, body, None)
        pltpu.sync_copy(acc, out_hbm.at[pl.ds(off, MSL)])

    return combine


_combine_max = _make_combine(jnp.maximum)
_combine_sum = _make_combine(lambda a, b: a + b)


@functools.partial(
    pl.kernel,
    out_type=(
        jax.ShapeDtypeStruct((H * E,), jnp.float32),
        jax.ShapeDtypeStruct((NW * MFLAT,), jnp.float32),
    ),
    mesh=_mesh,
    compiler_params=_sc_params,
    scratch_types=[
        pltpu.VMEM((C,), jnp.int32),
        pltpu.VMEM((H, C), jnp.float32),
        pltpu.VMEM((H, C), jnp.float32),
        pltpu.VMEM((MFLAT,), jnp.float32),
        pltpu.VMEM((MFLAT,), jnp.float32),
    ],
)
def _pass_a2(l_hbm, m_hbm, dst_hbm, e_hbm, sp_hbm,
             idx_d, lbuf, ebuf, m0, s_tile):
    cid = lax.axis_index("c")
    sid = lax.axis_index("s")
    wid = cid * NT + sid

    pltpu.sync_copy(m_hbm, m0)
    _fill_flat(s_tile, MFLAT, 0.0)

    iota = _iota16()

    def accumulate(flat, ev):
        def cond(mask):
            return jnp.any(mask)

        def body(mask):
            unused_cnt, last = plsc.scan_count(flat, mask)
            commit = jnp.logical_and(mask, last)
            old = plsc.load_gather(s_tile, [flat])
            plsc.store_scatter(s_tile, [flat], old + ev, mask=commit)
            return jnp.logical_and(mask, jnp.logical_not(commit))

        lax.while_loop(cond, body, jnp.full((16,), True, jnp.bool_))

    def do_chunk(base, csz):
        ng = csz // 16
        pltpu.sync_copy(dst_hbm.at[pl.ds(base, csz)], idx_d.at[pl.ds(0, csz)])
        for h in range(H):
            pltpu.sync_copy(l_hbm.at[pl.ds(h * E + base, csz)],
                            lbuf.at[h, pl.ds(0, csz)])

        def group(g, carry):
            rows = g * 16 + iota
            dstv = idx_d[pl.ds(g * 16, 16)]
            for h in range(H):
                hv = jnp.full((16,), h, jnp.int32)
                lv = plsc.load_gather(lbuf, [hv, rows])
                flat = dstv * H + h
                mv = plsc.load_gather(m0, [flat])
                ev = jnp.exp(lv - mv)
                plsc.store_scatter(ebuf, [hv, rows], ev)
                accumulate(flat, ev)
            return carry

        lax.fori_loop(0, ng, group, None)
        for h in range(H):
            pltpu.sync_copy(ebuf.at[h, pl.ds(0, csz)],
                            e_hbm.at[pl.ds(h * E + base, csz)])

    def chunk_body(i, carry):
        do_chunk(wid * EW + i * C, C)
        return carry

    lax.fori_loop(0, EW // C, chunk_body, None)
    do_chunk(wid * EW + (EW // C) * C, A_TAIL)

    pltpu.sync_copy(s_tile, sp_hbm.at[pl.ds(wid * MFLAT, MFLAT)])


BP = 256
NPAIR = ES // BP
B_TAIL2 = ES - NPAIR * BP


@functools.partial(
    pl.kernel,
    out_type=(
        jax.ShapeDtypeStruct((H, NPAD, D), jnp.float32),
        jax.ShapeDtypeStruct((H, NPAD, D), jnp.float32),
    ),
    mesh=_mesh,
    compiler_params=_sc_params,
    scratch_types=[
        pltpu.VMEM((BP,), jnp.int32),
        pltpu.VMEM((BP,), jnp.int32),
        pltpu.VMEM((BP,), jnp.float32),
        pltpu.VMEM((C,), jnp.int32),
        pltpu.VMEM((C,), jnp.int32),
        pltpu.VMEM((C,), jnp.int32),
        pltpu.VMEM((C,), jnp.int32),
        pltpu.VMEM((B_TAIL,), jnp.int32),
        pltpu.VMEM((C, D), jnp.float32),
        pltpu.VMEM((C, D), jnp.float32),
        pltpu.VMEM_SHARED((NPAD, D), jnp.float32),
        pltpu.SemaphoreType.DMA,
        pltpu.SemaphoreType.DMA,
    ],
)
def _pass_b(x0_hbm, src0_hbm, dst0_hbm, e0_hbm,
            x1_hbm, src1_hbm, dst1_hbm, e1_hbm, u0_hbm, u1_hbm,
            sbig, dbig, ebig, idxw0, idxw1, idxg0, idxg1, idxwt,
            xbuf0, xbuf1, u_shared, gsem0, gsem1):
    cid = lax.axis_index("c")
    sid = lax.axis_index("s")
    iota = _iota16()
    zi = jnp.zeros((16,), jnp.int32)

    def compute_chunk(xbuf, idxw, off):
        def cpidx(k, carry):
            idxw[pl.ds(k * 16, 16)] = dbig[pl.ds(off + k * 16, 16)]
            return carry

        lax.fori_loop(0, C // 16, cpidx, None)

        def group(g, carry):
            wv = ebig[pl.ds(off + g * 16, 16)]
            for lane in range(16):
                w = wv[lane]
                r = g * 16 + lane
                for j in range(D // 16):
                    sl = pl.ds(j * 16, 16)
                    xbuf[r, sl] = xbuf[r, sl] * w
            return carry

        lax.fori_loop(0, C // 16, group, None)

    for x_hbm, src_hbm, dst_hbm, e_hbm, u_hbm in (
            (x0_hbm, src0_hbm, dst0_hbm, e0_hbm, u0_hbm),
            (x1_hbm, src1_hbm, dst1_hbm, e1_hbm, u1_hbm)):
        for ph in range(2):
            h = 2 * ph + cid
            _zero_rows(xbuf0, C, D)
            for z in range(UROWS // C):
                pltpu.sync_copy(xbuf0,
                                u_shared.at[pl.ds(sid * UROWS + z * C, C)])
            plsc.subcore_barrier()

            def pair_body(i, carry, src_hbm=src_hbm, dst_hbm=dst_hbm,
                          e_hbm=e_hbm, x_hbm=x_hbm, h=h):
                base = sid * ES + i * BP
                pltpu.sync_copy(src_hbm.at[pl.ds(base, BP)], sbig)
                pltpu.sync_copy(dst_hbm.at[pl.ds(base, BP)], dbig)
                pltpu.sync_copy(e_hbm.at[pl.ds(h * E + base, BP)], ebig)

                def cpidx(k, carry):
                    idxg0[pl.ds(k * 16, 16)] = sbig[pl.ds(k * 16, 16)]
                    idxg1[pl.ds(k * 16, 16)] = sbig[pl.ds(C + k * 16, 16)]
                    return carry

                lax.fori_loop(0, C // 16, cpidx, None)
                g0 = pltpu.async_copy(x_hbm.at[idxg0], xbuf0, gsem0)
                g1 = pltpu.async_copy(x_hbm.at[idxg1], xbuf1, gsem1)
                g0.wait()
                compute_chunk(xbuf0, idxw0, 0)
                pltpu.sync_copy(xbuf0, u_shared.at[idxw0], add=True)
                g1.wait()
                compute_chunk(xbuf1, idxw1, C)
                pltpu.sync_copy(xbuf1, u_shared.at[idxw1], add=True)
                return carry

            lax.fori_loop(0, NPAIR, pair_body, None)

            tbase = sid * ES + NPAIR * BP
            pltpu.sync_copy(src_hbm.at[pl.ds(tbase, B_TAIL2)],
                            sbig.at[pl.ds(0, B_TAIL2)])
            pltpu.sync_copy(dst_hbm.at[pl.ds(tbase, B_TAIL2)], idxwt)
            pltpu.sync_copy(e_hbm.at[pl.ds(h * E + tbase, B_TAIL2)],
                            ebig.at[pl.ds(0, B_TAIL2)])
            for k in range(B_TAIL2 // 16):
                idxg0[pl.ds(k * 16, 16)] = sbig[pl.ds(k * 16, 16)]
            pltpu.async_copy(x_hbm.at[idxg0.at[pl.ds(0, B_TAIL2)]],
                             xbuf0.at[pl.ds(0, B_TAIL2)], gsem0).wait()

            def tail_group(g, carry):
                wv = ebig[pl.ds(g * 16, 16)]
                for lane in range(16):
                    w = wv[lane]
                    r = g * 16 + lane
                    for j in range(D // 16):
                        sl = pl.ds(j * 16, 16)
                        xbuf0[r, sl] = xbuf0[r, sl] * w
                return carry

            lax.fori_loop(0, B_TAIL2 // 16, tail_group, None)
            pltpu.sync_copy(xbuf0.at[pl.ds(0, B_TAIL2)], u_shared.at[idxwt],
                            add=True)

            plsc.subcore_barrier()
            for z in range(UROWS // C):
                r = sid * UROWS + z * C
                pltpu.sync_copy(u_shared.at[pl.ds(r, C)],
                                u_hbm.at[h, pl.ds(r, C)])
            plsc.subcore_barrier()


def _qk_body(xu_ref, xi_ref, wqu_ref, wki_ref, wqi_ref, wku_ref,
             qu_ref, ki_ref, qi_ref, ku_ref):
    xu = xu_ref[...]
    xi = xi_ref[...]
    qu_ref[...] = jnp.dot(xu, wqu_ref[...], preferred_element_type=jnp.float32)
    ki_ref[...] = jnp.dot(xi, wki_ref[...], preferred_element_type=jnp.float32)
    qi_ref[...] = jnp.dot(xi, wqi_ref[...], preferred_element_type=jnp.float32)
    ku_ref[...] = jnp.dot(xu, wku_ref[...], preferred_element_type=jnp.float32)


_QK_BLK = 2000


def _tc_qk(x_user, x_item, wqu, wki, wqi, wku):
    n_blk = N // _QK_BLK
    row_spec = pl.BlockSpec((_QK_BLK, D), lambda i: (i, 0))
    w_spec = pl.BlockSpec((D, D), lambda i: (0, 0))
    out = jax.ShapeDtypeStruct((N, D), jnp.float32)
    return pl.pallas_call(
        _qk_body,
        grid=(n_blk,),
        in_specs=[row_spec, row_spec, w_spec, w_spec, w_spec, w_spec],
        out_specs=[row_spec] * 4,
        out_shape=[out] * 4,
    )(x_user, x_item, wqu, wki, wqi, wku)


def _finish_body(u_ref, s_ref, wv_ref, o_ref):
    acc = jnp.zeros(o_ref.shape, jnp.float32)
    for h in range(H):
        u = u_ref[h]
        sh = s_ref[:, h:h + 1]
        zp = u / (sh + 1e-9)
        acc = acc + jax.nn.relu(
            jnp.dot(zp, wv_ref[h], preferred_element_type=jnp.float32))
    o_ref[...] = acc * (1.0 / H)


_FIN_BLK = 2000


def _tc_finish(u, s, wv):
    n_blk = N // _FIN_BLK
    return pl.pallas_call(
        _finish_body,
        grid=(n_blk,),
        in_specs=[
            pl.BlockSpec((H, _FIN_BLK, D), lambda i: (0, i, 0)),
            pl.BlockSpec((_FIN_BLK, H), lambda i: (i, 0)),
            pl.BlockSpec((H, D, D), lambda i: (0, 0, 0)),
        ],
        out_specs=pl.BlockSpec((_FIN_BLK, D), lambda i: (i, 0)),
        out_shape=jax.ShapeDtypeStruct((N, D), jnp.float32),
    )(u, s, wv)


def _edge_stages(q, k, src, dst):
    lh, mp = _pass_a1(q, k, src, dst)
    m = _combine_max(mp)
    eh, sp = _pass_a2(lh, m, dst)
    s = _combine_sum(sp)
    return eh, s.reshape(NPAD, H)


def _cat_heads(w):
    return jnp.transpose(w, (1, 0, 2)).reshape(D, H * DH)


@jax.jit
def kernel(u2i, i2u, x_user, x_item, user_w_q, user_w_k, user_w_v,
           item_w_q, item_w_k, item_w_v):
    wqu = _cat_heads(user_w_q)
    wki = _cat_heads(item_w_k)
    wqi = _cat_heads(item_w_q)
    wku = _cat_heads(user_w_k)
    qu, ki, qi, ku = _tc_qk(x_user, x_item, wqu, wki, wqi, wku)

    eh_u, s_u = _edge_stages(qu, ki, i2u[0], i2u[1])
    eh_i, s_i = _edge_stages(qi, ku, u2i[0], u2i[1])

    u_u, u_i = _pass_b(x_item, i2u[0], i2u[1], eh_u,
                       x_user, u2i[0], u2i[1], eh_i)

    z_user = _tc_finish(u_u, s_u, item_w_v)
    z_item = _tc_finish(u_i, s_i, user_w_v)
    return (z_user, z_item)
